# baseline (speedup 1.0000x reference)
import functools

import jax
import jax.numpy as jnp
from jax import lax
from jax.experimental import pallas as pl
from jax.experimental.pallas import tpu as pltpu
from jax.experimental.pallas import tpu_sc as plsc

BATCH = 16384
EMBED_DIM = 32
NUM_CORES = 2
NUM_SUBCORES = 16
LANES = 16
NUM_WORKERS = NUM_CORES * NUM_SUBCORES
BPW = BATCH // NUM_WORKERS
CHUNK = 128
NUM_CHUNKS = BPW // CHUNK
GROUPS = BPW // LANES

_mesh = plsc.VectorSubcoreMesh(core_axis_name="c", subcore_axis_name="s")


@functools.partial(
    pl.kernel,
    out_type=jax.ShapeDtypeStruct((BATCH,), jnp.float32),
    mesh=_mesh,
    compiler_params=pltpu.CompilerParams(
        needs_layout_passes=False, use_tc_tiling_on_sc=False
    ),
    scratch_types=[
        pltpu.VMEM((BPW,), jnp.int32),
        pltpu.VMEM((BPW,), jnp.int32),
        pltpu.VMEM((BPW, EMBED_DIM), jnp.float32),
        pltpu.VMEM((BPW, EMBED_DIM), jnp.float32),
        pltpu.VMEM((BPW,), jnp.float32),
        pltpu.SemaphoreType.DMA,
        pltpu.SemaphoreType.DMA,
    ],
)
def _sc_dot(uid_hbm, iid_hbm, utab_hbm, itab_hbm, out_hbm,
            uidx, iidx, urows, irows, outb, sem_u, sem_i):
    wid = lax.axis_index("s") * NUM_CORES + lax.axis_index("c")
    base = wid * BPW

    pltpu.sync_copy(uid_hbm.at[pl.ds(base, BPW)], uidx)
    pltpu.sync_copy(iid_hbm.at[pl.ds(base, BPW)], iidx)

    def chunk_views(j):
        sl = pl.ds(j * CHUNK, CHUNK)
        return (utab_hbm.at[uidx.at[sl]], urows.at[sl],
                itab_hbm.at[iidx.at[sl]], irows.at[sl])

    for j in range(NUM_CHUNKS):
        usrc, udst, isrc, idst = chunk_views(j)
        pltpu.async_copy(usrc, udst, sem_u)
        pltpu.async_copy(isrc, idst, sem_i)
    for j in range(NUM_CHUNKS):
        usrc, udst, isrc, idst = chunk_views(j)
        pltpu.make_async_copy(usrc, udst, sem_u).wait()
        pltpu.make_async_copy(isrc, idst, sem_i).wait()

    lane = lax.iota(jnp.int32, LANES)

    def group_body(g, carry):
        rvec = g * LANES + lane
        acc = jnp.zeros((LANES,), jnp.float32)
        for d in range(EMBED_DIM):
            col = jnp.full((LANES,), d, jnp.int32)
            uu = plsc.load_gather(urows, [rvec, col])
            vv = plsc.load_gather(irows, [rvec, col])
            acc = acc + uu * vv
        outb[pl.ds(g * LANES, LANES)] = acc
        return carry

    lax.fori_loop(0, GROUPS, group_body, 0)

    pltpu.sync_copy(outb, out_hbm.at[pl.ds(base, BPW)])


def kernel(user_ids, item_ids, user_table, item_table):
    return _sc_dot(user_ids.astype(jnp.int32), item_ids.astype(jnp.int32),
                   user_table, item_table)
